# Initial kernel scaffold; baseline (speedup 1.0000x reference)
import jax
import jax.numpy as jnp
from jax.experimental import pallas as pl


def kernel(input_ids, table, W, b):
    raise NotImplementedError("write your pallas kernel here")


# speedup vs baseline: 74.6915x; 74.6915x over previous
import functools

import jax
import jax.numpy as jnp
from jax import lax
from jax.experimental import pallas as pl
from jax.experimental.pallas import tpu as pltpu
from jax.experimental.pallas import tpu_sc as plsc

VOCAB = 30522
D = 768
NCLS = 2
B = 4096
L = 200

BLK = 2048
VP = 30720

NC = 2
NS = 16
NW = NC * NS
SEQ_PER_W = B // NW
GROUPS = SEQ_PER_W // 16


def _tw_body(w_ref, tbl_ref, out_ref):
    out_ref[...] = lax.dot_general(
        w_ref[...], tbl_ref[...],
        dimension_numbers=(((1,), (1,)), ((), ())),
        preferred_element_type=jnp.float32,
    )


def _compute_tw(table, W):
    return pl.pallas_call(
        _tw_body,
        grid=(VP // BLK,),
        in_specs=[
            pl.BlockSpec((NCLS, D), lambda i: (0, 0)),
            pl.BlockSpec((BLK, D), lambda i: (i, 0)),
        ],
        out_specs=pl.BlockSpec((NCLS, BLK), lambda i: (0, i)),
        out_shape=jax.ShapeDtypeStruct((NCLS, VP), jnp.float32),
    )(W, table)


def _sc_kernel(tw_hbm, ids_hbm, bias_hbm, out0_hbm, out1_hbm,
               tw_v, ids_v, bias_v, out0_v, out1_v):
    wid = lax.axis_index("s") * NC + lax.axis_index("c")
    base = wid * SEQ_PER_W

    pltpu.sync_copy(ids_hbm.at[wid], ids_v)
    pltpu.sync_copy(tw_hbm, tw_v)
    pltpu.sync_copy(bias_hbm, bias_v)

    zero = jnp.zeros((16,), jnp.float32)

    def body(t, accs):
        new = []
        for g in range(GROUPS):
            idx = ids_v[t, pl.ds(g * 16, 16)]
            v0 = plsc.load_gather(tw_v, [idx])
            v1 = plsc.load_gather(tw_v, [idx + VP])
            new.append(accs[2 * g] + v0)
            new.append(accs[2 * g + 1] + v1)
        return tuple(new)

    accs = lax.fori_loop(0, L, body, (zero,) * (2 * GROUPS))

    inv_l = jnp.float32(1.0 / L)
    bvec = bias_v[...]
    b0 = bvec[0]
    b1 = bvec[1]
    for g in range(GROUPS):
        out0_v[pl.ds(g * 16, 16)] = accs[2 * g] * inv_l + b0
        out1_v[pl.ds(g * 16, 16)] = accs[2 * g + 1] * inv_l + b1

    pltpu.sync_copy(out0_v, out0_hbm.at[pl.ds(base, SEQ_PER_W)])
    pltpu.sync_copy(out1_v, out1_hbm.at[pl.ds(base, SEQ_PER_W)])


def _pool_logits(tw_flat, ids_w, bias16):
    mesh = plsc.VectorSubcoreMesh(core_axis_name="c", subcore_axis_name="s")
    f = functools.partial(
        pl.kernel,
        mesh=mesh,
        out_type=(
            jax.ShapeDtypeStruct((B,), jnp.float32),
            jax.ShapeDtypeStruct((B,), jnp.float32),
        ),
        scratch_types=[
            pltpu.VMEM((2 * VP,), jnp.float32),
            pltpu.VMEM((L, SEQ_PER_W), jnp.int32),
            pltpu.VMEM((16,), jnp.float32),
            pltpu.VMEM((SEQ_PER_W,), jnp.float32),
            pltpu.VMEM((SEQ_PER_W,), jnp.float32),
        ],
        compiler_params=pltpu.CompilerParams(needs_layout_passes=False),
    )(_sc_kernel)
    return f(tw_flat, ids_w, bias16)


def kernel(input_ids, table, W, b):
    tw = _compute_tw(table, W)
    tw_flat = tw.reshape(2 * VP)

    ids = input_ids.astype(jnp.int32)
    ids_w = ids.reshape(NW, SEQ_PER_W, L).transpose(0, 2, 1)

    bias16 = jnp.pad(b.astype(jnp.float32), (0, 16 - NCLS))

    out0, out1 = _pool_logits(tw_flat, ids_w, bias16)
    return jnp.stack([out0, out1], axis=-1)
